# Initial kernel scaffold; baseline (speedup 1.0000x reference)
import jax
import jax.numpy as jnp
from jax.experimental import pallas as pl


def kernel(hidden_states, moe_mm_token_type_ids, text_router_w, text_bias, text_gate_up, text_down, vis_router_w, vis_bias, vis_gate_up, vis_down, shared_gate_w, shared_up_w, shared_down_w):
    raise NotImplementedError("write your pallas kernel here")


# speedup vs baseline: 2.2132x; 2.2132x over previous
import functools

import jax
import jax.numpy as jnp
from jax.experimental import pallas as pl
from jax.experimental.pallas import tpu as pltpu

_B, _S = 8, 4
_T = _B * _S
_H = 1024
_E = 64
_FF_TEXT = 1024
_FF_VIS = 512
_SH = 2048
_NORM_MIN = 1e-12
_NEG = -1e30


def _dotT(a, b):
    return jax.lax.dot_general(a, b, (((1,), (1,)), ((), ())),
                               preferred_element_type=jnp.float32)


def _routing_body(x_ref, tw_ref, vw_ref, tb_ref, vb_ref, tt_ref,
                  rl_ref, comb_t_ref, comb_v_ref, sched_t_ref, sched_v_ref):
    x = x_ref[...]
    ttcol = tt_ref[...]
    lane_e = jax.lax.broadcasted_iota(jnp.int32, (_T, _E), 1).astype(jnp.float32)

    logits_t = _dotT(x, tw_ref[...])
    logits_v = _dotT(x, vw_ref[...])

    is_vis = (ttcol != 0)
    rl_ref[...] = jnp.where(is_vis, logits_v, logits_t)

    def one_modality(logits, bias_row, mod_mask_col):
        probs = jax.nn.softmax(logits, axis=1)
        corrected = probs + bias_row
        m1 = jnp.max(corrected, axis=1, keepdims=True)
        i1 = jnp.min(jnp.where(corrected >= m1, lane_e, 1e9), axis=1,
                     keepdims=True)
        eq1 = (lane_e == i1).astype(jnp.float32)
        p1 = jnp.sum(probs * eq1, axis=1, keepdims=True)
        c2 = jnp.where(eq1 > 0, _NEG, corrected)
        m2 = jnp.max(c2, axis=1, keepdims=True)
        i2 = jnp.min(jnp.where(c2 >= m2, lane_e, 1e9), axis=1, keepdims=True)
        eq2 = (lane_e == i2).astype(jnp.float32)
        p2 = jnp.sum(probs * eq2, axis=1, keepdims=True)

        denom = jnp.maximum(p1 + p2, _NORM_MIN)
        w1 = p1 / denom
        w2 = p2 / denom
        comb = (eq1 * w1 + eq2 * w2) * mod_mask_col

        presence = (eq1 + eq2) * mod_mask_col
        sel = (jnp.sum(presence, axis=0, keepdims=True) > 0).astype(jnp.float32)

        ii = jax.lax.broadcasted_iota(jnp.int32, (_E, _E), 0).astype(jnp.float32)
        jj = jax.lax.broadcasted_iota(jnp.int32, (_E, _E), 1).astype(jnp.float32)
        lt = (jj <= ii).astype(jnp.float32)
        rank = jax.lax.dot_general(sel, lt, (((1,), (1,)), ((), ())),
                                   preferred_element_type=jnp.float32)
        n = jnp.max(rank)
        m3 = ((rank == (ii + 1.0)).astype(jnp.float32) * sel)
        iota_col = jax.lax.broadcasted_iota(jnp.int32, (_E, 1), 0).astype(jnp.float32)
        sched0 = jax.lax.dot_general(m3, iota_col, (((1,), (0,)), ((), ())),
                                     preferred_element_type=jnp.float32)
        cm = jnp.minimum(iota_col, jnp.maximum(n, 1.0) - 1.0)
        b2 = (jj == cm).astype(jnp.float32)
        sched = jax.lax.dot_general(b2, sched0, (((1,), (0,)), ((), ())),
                                    preferred_element_type=jnp.float32)
        return comb, sched

    mt = (ttcol == 0).astype(jnp.float32)
    mv = (ttcol != 0).astype(jnp.float32)
    comb_t, sched_t = one_modality(logits_t, tb_ref[...], mt)
    comb_v, sched_v = one_modality(logits_v, vb_ref[...], mv)
    comb_t_ref[...] = comb_t
    comb_v_ref[...] = comb_v
    sched_t_ref[...] = sched_t
    sched_v_ref[...] = sched_v


def _shared_body(x_ref, gw_ref, uw_ref, dw_ref, out_ref):
    i = pl.program_id(0)

    @pl.when(i == 0)
    def _():
        out_ref[...] = jnp.zeros_like(out_ref)

    x = x_ref[...]
    g = _dotT(x, gw_ref[...])
    u = _dotT(x, uw_ref[...])
    h = g * jax.nn.sigmoid(g) * u
    out_ref[...] += _dotT(h, dw_ref[...])


def _expert_body(sched_ref, x_ref, gu_ref, dn_ref, comb_ref, base_ref, out_ref,
                 *, ff):
    i = pl.program_id(0)
    e = sched_ref[i]
    ep = sched_ref[jnp.maximum(i - 1, 0)]
    fresh = jnp.logical_or(i == 0, e != ep)

    @pl.when(i == 0)
    def _():
        out_ref[...] = base_ref[...]

    @pl.when(fresh)
    def _():
        x = x_ref[...]
        gu = jnp.dot(x, gu_ref[0], preferred_element_type=jnp.float32)
        g = gu[:, :ff]
        u = gu[:, ff:]
        h = g * jax.nn.sigmoid(g) * u
        y = jnp.dot(h, dn_ref[0], preferred_element_type=jnp.float32)
        lane = jax.lax.broadcasted_iota(jnp.int32, (_T, _E), 1)
        crow = jnp.sum(jnp.where(lane == e, comb_ref[...], 0.0), axis=1,
                       keepdims=True)
        out_ref[...] += y * crow


def _expert_call(sched, x, gate_up, down, comb, base, ff):
    grid_spec = pltpu.PrefetchScalarGridSpec(
        num_scalar_prefetch=1,
        grid=(_E,),
        in_specs=[
            pl.BlockSpec((_T, _H), lambda i, s: (0, 0)),
            pl.BlockSpec((1, _H, 2 * ff), lambda i, s: (s[i], 0, 0)),
            pl.BlockSpec((1, ff, _H), lambda i, s: (s[i], 0, 0)),
            pl.BlockSpec((_T, _E), lambda i, s: (0, 0)),
            pl.BlockSpec((_T, _H), lambda i, s: (0, 0)),
        ],
        out_specs=pl.BlockSpec((_T, _H), lambda i, s: (0, 0)),
    )
    return pl.pallas_call(
        functools.partial(_expert_body, ff=ff),
        grid_spec=grid_spec,
        out_shape=jax.ShapeDtypeStruct((_T, _H), jnp.float32),
    )(sched, x, gate_up, down, comb, base)


def kernel(hidden_states, moe_mm_token_type_ids, text_router_w, text_bias,
           text_gate_up, text_down, vis_router_w, vis_bias, vis_gate_up,
           vis_down, shared_gate_w, shared_up_w, shared_down_w):
    Bv, Sv, D = hidden_states.shape
    x = hidden_states.reshape(-1, D)
    ttcol = moe_mm_token_type_ids.reshape(-1, 1).astype(jnp.int32)

    rl, comb_t, comb_v, sched_t_f, sched_v_f = pl.pallas_call(
        _routing_body,
        out_shape=[
            jax.ShapeDtypeStruct((_T, _E), jnp.float32),
            jax.ShapeDtypeStruct((_T, _E), jnp.float32),
            jax.ShapeDtypeStruct((_T, _E), jnp.float32),
            jax.ShapeDtypeStruct((_E, 1), jnp.float32),
            jax.ShapeDtypeStruct((_E, 1), jnp.float32),
        ],
    )(x, text_router_w, vis_router_w, text_bias.reshape(1, _E),
      vis_bias.reshape(1, _E), ttcol)

    sched_t = sched_t_f.reshape(_E).astype(jnp.int32)
    sched_v = sched_v_f.reshape(_E).astype(jnp.int32)

    n_chunks = 4
    chunk = _SH // n_chunks
    shared = pl.pallas_call(
        _shared_body,
        grid=(n_chunks,),
        in_specs=[
            pl.BlockSpec((_T, _H), lambda i: (0, 0)),
            pl.BlockSpec((chunk, _H), lambda i: (i, 0)),
            pl.BlockSpec((chunk, _H), lambda i: (i, 0)),
            pl.BlockSpec((_H, chunk), lambda i: (0, i)),
        ],
        out_specs=pl.BlockSpec((_T, _H), lambda i: (0, 0)),
        out_shape=jax.ShapeDtypeStruct((_T, _H), jnp.float32),
    )(x, shared_gate_w, shared_up_w, shared_down_w)

    acc = _expert_call(sched_t, x, text_gate_up, text_down, comb_t, shared,
                       _FF_TEXT)
    final = _expert_call(sched_v, x, vis_gate_up, vis_down, comb_v, acc,
                         _FF_VIS)

    return final.reshape(Bv, Sv, D), rl


# speedup vs baseline: 2.2134x; 1.0001x over previous
import functools

import jax
import jax.numpy as jnp
from jax.experimental import pallas as pl
from jax.experimental.pallas import tpu as pltpu

_B, _S = 8, 4
_T = _B * _S
_H = 1024
_E = 64
_FF_TEXT = 1024
_FF_VIS = 512
_SH = 2048
_NORM_MIN = 1e-12
_NEG = -1e30


def _dotT(a, b):
    return jax.lax.dot_general(a, b, (((1,), (1,)), ((), ())),
                               preferred_element_type=jnp.float32)


def _routing_body(x_ref, tw_ref, vw_ref, tb_ref, vb_ref, tt_ref,
                  rl_ref, comb_t_ref, comb_v_ref, sched_t_ref, sched_v_ref):
    x = x_ref[...]
    ttcol = tt_ref[...]
    lane_e = jax.lax.broadcasted_iota(jnp.int32, (_T, _E), 1).astype(jnp.float32)

    logits_t = _dotT(x, tw_ref[...])
    logits_v = _dotT(x, vw_ref[...])

    is_vis = (ttcol != 0)
    rl_ref[...] = jnp.where(is_vis, logits_v, logits_t)

    def one_modality(logits, bias_row, mod_mask_col):
        probs = jax.nn.softmax(logits, axis=1)
        corrected = probs + bias_row
        m1 = jnp.max(corrected, axis=1, keepdims=True)
        i1 = jnp.min(jnp.where(corrected >= m1, lane_e, 1e9), axis=1,
                     keepdims=True)
        eq1 = (lane_e == i1).astype(jnp.float32)
        p1 = jnp.sum(probs * eq1, axis=1, keepdims=True)
        c2 = jnp.where(eq1 > 0, _NEG, corrected)
        m2 = jnp.max(c2, axis=1, keepdims=True)
        i2 = jnp.min(jnp.where(c2 >= m2, lane_e, 1e9), axis=1, keepdims=True)
        eq2 = (lane_e == i2).astype(jnp.float32)
        p2 = jnp.sum(probs * eq2, axis=1, keepdims=True)

        denom = jnp.maximum(p1 + p2, _NORM_MIN)
        w1 = p1 / denom
        w2 = p2 / denom
        comb = (eq1 * w1 + eq2 * w2) * mod_mask_col

        presence = (eq1 + eq2) * mod_mask_col
        sel = (jnp.sum(presence, axis=0, keepdims=True) > 0).astype(jnp.float32)

        ii = jax.lax.broadcasted_iota(jnp.int32, (_E, _E), 0).astype(jnp.float32)
        jj = jax.lax.broadcasted_iota(jnp.int32, (_E, _E), 1).astype(jnp.float32)
        lt = (jj <= ii).astype(jnp.float32)
        rank = jax.lax.dot_general(sel, lt, (((1,), (1,)), ((), ())),
                                   preferred_element_type=jnp.float32)
        n = jnp.max(rank)
        m3 = ((rank == (ii + 1.0)).astype(jnp.float32) * sel)
        iota_col = jax.lax.broadcasted_iota(jnp.int32, (_E, 1), 0).astype(jnp.float32)
        sched0 = jax.lax.dot_general(m3, iota_col, (((1,), (0,)), ((), ())),
                                     preferred_element_type=jnp.float32)
        cm = jnp.minimum(iota_col, jnp.maximum(n, 1.0) - 1.0)
        b2 = (jj == cm).astype(jnp.float32)
        sched = jax.lax.dot_general(b2, sched0, (((1,), (0,)), ((), ())),
                                    preferred_element_type=jnp.float32)
        return comb, sched

    mt = (ttcol == 0).astype(jnp.float32)
    mv = (ttcol != 0).astype(jnp.float32)
    comb_t, sched_t = one_modality(logits_t, tb_ref[...], mt)
    comb_v, sched_v = one_modality(logits_v, vb_ref[...], mv)
    comb_t_ref[...] = comb_t
    comb_v_ref[...] = comb_v
    sched_t_ref[...] = sched_t
    sched_v_ref[...] = sched_v


def _shared_body(x_ref, gw_ref, uw_ref, dw_ref, out_ref):
    i = pl.program_id(0)

    @pl.when(i == 0)
    def _():
        out_ref[...] = jnp.zeros_like(out_ref)

    x = x_ref[...]
    g = _dotT(x, gw_ref[...])
    u = _dotT(x, uw_ref[...])
    h = g * jax.nn.sigmoid(g) * u
    out_ref[...] += _dotT(h, dw_ref[...])


def _expert_body(sched_ref, x_ref, g_ref, u_ref, dn_ref, comb_ref, base_ref,
                 out_ref):
    i = pl.program_id(0)
    e = sched_ref[i]
    ep = sched_ref[jnp.maximum(i - 1, 0)]
    fresh = jnp.logical_or(i == 0, e != ep)

    @pl.when(i == 0)
    def _():
        out_ref[...] = base_ref[...]

    @pl.when(fresh)
    def _():
        x = x_ref[...]
        g = jnp.dot(x, g_ref[0], preferred_element_type=jnp.float32)
        u = jnp.dot(x, u_ref[0], preferred_element_type=jnp.float32)
        h = g * jax.nn.sigmoid(g) * u
        y = jnp.dot(h, dn_ref[0], preferred_element_type=jnp.float32)
        lane = jax.lax.broadcasted_iota(jnp.int32, (_T, _E), 1)
        crow = jnp.sum(jnp.where(lane == e, comb_ref[...], 0.0), axis=1,
                       keepdims=True)
        out_ref[...] += y * crow


def _expert_call(sched, x, gate_up, down, comb, base, ff):
    grid_spec = pltpu.PrefetchScalarGridSpec(
        num_scalar_prefetch=1,
        grid=(_E,),
        in_specs=[
            pl.BlockSpec((_T, _H), lambda i, s: (0, 0)),
            pl.BlockSpec((1, _H, ff), lambda i, s: (s[i], 0, 0)),
            pl.BlockSpec((1, _H, ff), lambda i, s: (s[i], 0, 1)),
            pl.BlockSpec((1, ff, _H), lambda i, s: (s[i], 0, 0)),
            pl.BlockSpec((_T, _E), lambda i, s: (0, 0)),
            pl.BlockSpec((_T, _H), lambda i, s: (0, 0)),
        ],
        out_specs=pl.BlockSpec((_T, _H), lambda i, s: (0, 0)),
    )
    return pl.pallas_call(
        _expert_body,
        grid_spec=grid_spec,
        out_shape=jax.ShapeDtypeStruct((_T, _H), jnp.float32),
    )(sched, x, gate_up, gate_up, down, comb, base)


def kernel(hidden_states, moe_mm_token_type_ids, text_router_w, text_bias,
           text_gate_up, text_down, vis_router_w, vis_bias, vis_gate_up,
           vis_down, shared_gate_w, shared_up_w, shared_down_w):
    Bv, Sv, D = hidden_states.shape
    x = hidden_states.reshape(-1, D)
    ttcol = moe_mm_token_type_ids.reshape(-1, 1).astype(jnp.int32)

    rl, comb_t, comb_v, sched_t_f, sched_v_f = pl.pallas_call(
        _routing_body,
        out_shape=[
            jax.ShapeDtypeStruct((_T, _E), jnp.float32),
            jax.ShapeDtypeStruct((_T, _E), jnp.float32),
            jax.ShapeDtypeStruct((_T, _E), jnp.float32),
            jax.ShapeDtypeStruct((_E, 1), jnp.float32),
            jax.ShapeDtypeStruct((_E, 1), jnp.float32),
        ],
    )(x, text_router_w, vis_router_w, text_bias.reshape(1, _E),
      vis_bias.reshape(1, _E), ttcol)

    sched_t = sched_t_f.reshape(_E).astype(jnp.int32)
    sched_v = sched_v_f.reshape(_E).astype(jnp.int32)

    n_chunks = 4
    chunk = _SH // n_chunks
    shared = pl.pallas_call(
        _shared_body,
        grid=(n_chunks,),
        in_specs=[
            pl.BlockSpec((_T, _H), lambda i: (0, 0)),
            pl.BlockSpec((chunk, _H), lambda i: (i, 0)),
            pl.BlockSpec((chunk, _H), lambda i: (i, 0)),
            pl.BlockSpec((_H, chunk), lambda i: (0, i)),
        ],
        out_specs=pl.BlockSpec((_T, _H), lambda i: (0, 0)),
        out_shape=jax.ShapeDtypeStruct((_T, _H), jnp.float32),
    )(x, shared_gate_w, shared_up_w, shared_down_w)

    acc = _expert_call(sched_t, x, text_gate_up, text_down, comb_t, shared,
                       _FF_TEXT)
    final = _expert_call(sched_v, x, vis_gate_up, vis_down, comb_v, acc,
                         _FF_VIS)

    return final.reshape(Bv, Sv, D), rl
